# baseline (speedup 1.0000x reference)
import functools

import jax
import jax.numpy as jnp
from jax import lax
from jax.experimental import pallas as pl
from jax.experimental.pallas import tpu as pltpu
from jax.experimental.pallas import tpu_sc as plsc

N = 10000
NP = 10240
E = 160000
CHUNK = 128
CPT = 80
ET = CPT * CHUNK
EP = 16 * ET
NC, NS = 2, 16
RPT = NP // NS
NPC = NP // NC
RPT2 = NPC // NS

_i32 = jnp.int32
_f32 = jnp.float32


def _rsqrt16(d):
    h = d * 0.5
    i = lax.bitcast_convert_type(d, _i32)
    i = jnp.full((16,), 0x5F3759DF, _i32) - lax.shift_right_logical(i, 1)
    y = lax.bitcast_convert_type(i, _f32)
    y = y * (1.5 - h * y * y)
    y = y * (1.5 - h * y * y)
    y = y * (1.5 - h * y * y)
    return y


def _vsplat(vec, lane):
    return jnp.take_along_axis(vec, jnp.full((16,), lane, _i32), axis=0,
                               mode="promise_in_bounds")


W1CH = 8


def _sc1_body(src_hbm, dst_hbm, ew_hbm, xg_hbm, agg_hbm, dinv_hbm,
              spm_agg, spm_deg, spm_dinv,
              src_w, dst_w, ew_w, dinv_v, dv, rows_a, rows_b,
              sem_a, sem_b, sem_sa, sem_sb):
    c = lax.axis_index("c")
    s = lax.axis_index("s")
    nw = CPT // W1CH

    def _z16(i, carry):
        dv[pl.ds(i * 16, 16)] = jnp.zeros((16,), _f32)
        return carry
    lax.fori_loop(0, 40, _z16, 0)
    pltpu.sync_copy(dv, spm_deg.at[pl.ds(s * 640, 640)])
    plsc.subcore_barrier()

    def _degw(w, carry):
        pltpu.sync_copy(dst_hbm.at[s, pl.ds(w * W1CH, W1CH)], dst_w)
        pltpu.sync_copy(ew_hbm.at[s, pl.ds(w * W1CH, W1CH)], ew_w)

        def _degj(j, carry2):
            pltpu.sync_copy(ew_w.at[j], spm_deg.at[dst_w.at[j]], add=True)
            return carry2
        lax.fori_loop(0, W1CH, _degj, 0)
        return carry
    lax.fori_loop(0, nw, _degw, 0)
    plsc.subcore_barrier()

    pltpu.sync_copy(spm_deg.at[pl.ds(s * 640, 640)], dv)

    def _dinvi(i, carry):
        d = dv[pl.ds(i * 16, 16)] + 1.0
        dv[pl.ds(i * 16, 16)] = _rsqrt16(d)
        return carry
    lax.fori_loop(0, 40, _dinvi, 0)
    pltpu.sync_copy(dv, spm_dinv.at[pl.ds(s * 640, 640)])

    @pl.when(c == 0)
    def _():
        pltpu.sync_copy(dv, dinv_hbm.at[pl.ds(s * 640, 640)])
    plsc.subcore_barrier()

    pltpu.sync_copy(spm_dinv, dinv_v)

    base = RPT * s
    for t in range(5):
        r0 = base + 128 * t
        pltpu.sync_copy(xg_hbm.at[pl.ds(NP * c + r0, 128)], rows_a)

        def _initr(r, carry):
            g = plsc.load_gather(dinv_v, [jnp.full((16,), r0, _i32) + r])
            g2 = g * g
            for f in range(8):
                rows_a[r, pl.ds(f * 16, 16)] = rows_a[r, pl.ds(f * 16, 16)] * g2
            return carry
        lax.fori_loop(0, 128, _initr, 0)
        pltpu.sync_copy(rows_a, spm_agg.at[pl.ds(r0, 128)])
    plsc.subcore_barrier()

    def _scale(rows, j):
        @plsc.parallel_loop(0, 8, unroll=2)
        def _sg(g):
            nm16 = ew_w[j, pl.ds(g * 16, 16)]
            for l in range(16):
                spl = _vsplat(nm16, l)
                e = g * 16 + l
                for f in range(8):
                    rows[e, pl.ds(f * 16, 16)] = rows[e, pl.ds(f * 16, 16)] * spl

    def _wait_scat(rows, sem):
        pltpu.make_async_copy(rows, spm_agg.at[dst_w.at[0]], sem).wait()

    def _gather(j, rows, sem):
        for q in range(4):
            pltpu.async_copy(xg_hbm.at[src_w.at[j, pl.ds(32 * q, 32)]],
                             rows.at[pl.ds(32 * q, 32)], sem)

    def _wait_gather(j, rows, sem):
        for q in range(4):
            pltpu.make_async_copy(xg_hbm.at[src_w.at[j, pl.ds(32 * q, 32)]],
                                  rows.at[pl.ds(32 * q, 32)], sem).wait()

    def _ew(w, carry):
        pltpu.sync_copy(src_hbm.at[s, pl.ds(w * W1CH, W1CH)], src_w)
        pltpu.sync_copy(dst_hbm.at[s, pl.ds(w * W1CH, W1CH)], dst_w)
        pltpu.sync_copy(ew_hbm.at[s, pl.ds(w * W1CH, W1CH)], ew_w)

        def _nk(k, carry2):
            jj = k // 8
            oo = (k % 8) * 16
            sv = src_w[jj, pl.ds(oo, 16)]
            dvv = dst_w[jj, pl.ds(oo, 16)]
            ev = ew_w[jj, pl.ds(oo, 16)]
            dsg = plsc.load_gather(dinv_v, [sv])
            ddg = plsc.load_gather(dinv_v, [dvv])
            ew_w[jj, pl.ds(oo, 16)] = ev * dsg * ddg
            src_w[jj, pl.ds(oo, 16)] = sv + NP * c
            return carry2
        lax.fori_loop(0, W1CH * 8, _nk, 0)

        _gather(0, rows_a, sem_a)
        _gather(1, rows_b, sem_b)

        def _ej(jj, carry2):
            j0 = 2 * jj
            j1 = 2 * jj + 1
            _wait_gather(j0, rows_a, sem_a)
            _scale(rows_a, j0)
            pltpu.async_copy(rows_a, spm_agg.at[dst_w.at[j0]], sem_sa, add=True)
            _wait_gather(j1, rows_b, sem_b)
            _scale(rows_b, j1)
            pltpu.async_copy(rows_b, spm_agg.at[dst_w.at[j1]], sem_sb, add=True)
            _wait_scat(rows_a, sem_sa)

            @pl.when(jj < W1CH // 2 - 1)
            def _():
                _gather(j0 + 2, rows_a, sem_a)
            _wait_scat(rows_b, sem_sb)

            @pl.when(jj < W1CH // 2 - 1)
            def _():
                _gather(j1 + 2, rows_b, sem_b)
            return carry2
        lax.fori_loop(0, W1CH // 2, _ej, 0)
        return carry
    lax.fori_loop(0, nw, _ew, 0)
    plsc.subcore_barrier()

    pltpu.sync_copy(spm_agg.at[pl.ds(base, RPT)],
                    agg_hbm.at[c, pl.ds(base, RPT)])


def _make_sc1():
    mesh = plsc.VectorSubcoreMesh(core_axis_name="c", subcore_axis_name="s")
    return pl.kernel(
        _sc1_body,
        out_type=(
            jax.ShapeDtypeStruct((NC, NP, 128), _f32),
            jax.ShapeDtypeStruct((NP,), _f32),
        ),
        mesh=mesh,
        compiler_params=pltpu.CompilerParams(needs_layout_passes=False),
        scratch_types=[
            pltpu.VMEM_SHARED((NP, 128), _f32),
            pltpu.VMEM_SHARED((NP,), _f32),
            pltpu.VMEM_SHARED((NP,), _f32),
            pltpu.VMEM((W1CH, CHUNK), _i32),
            pltpu.VMEM((W1CH, CHUNK), _i32),
            pltpu.VMEM((W1CH, CHUNK), _f32),
            pltpu.VMEM((NP,), _f32),
            pltpu.VMEM((640,), _f32),
            pltpu.VMEM((CHUNK, 128), _f32),
            pltpu.VMEM((CHUNK, 128), _f32),
            pltpu.SemaphoreType.DMA,
            pltpu.SemaphoreType.DMA,
            pltpu.SemaphoreType.DMA,
            pltpu.SemaphoreType.DMA,
        ],
    )


def _sc2_body(src_hbm, dst_hbm, ew_hbm, dinv_hbm, zt_hbm, b2_hbm, out_hbm,
              spm_c0, spm_c1, spm_c2,
              src_v, dst_v, ew_v, dinv_v, z0, z1, z2, msg_w, ab, ob, b2_v):
    c = lax.axis_index("c")
    s = lax.axis_index("s")
    g0 = NPC * c + RPT2 * s
    spm_cols = (spm_c0, spm_c1, spm_c2)
    z_cols = (z0, z1, z2)

    pltpu.sync_copy(src_hbm.at[s], src_v)
    pltpu.sync_copy(dst_hbm.at[s], dst_v)
    pltpu.sync_copy(ew_hbm.at[s], ew_v)
    pltpu.sync_copy(dinv_hbm, dinv_v)
    pltpu.sync_copy(b2_hbm, b2_v)
    for jc in range(3):
        pltpu.sync_copy(zt_hbm.at[pl.ds(jc * NP, NP)], z_cols[jc])

    def _z16(i, carry):
        ob[pl.ds(i * 16, 16)] = jnp.zeros((16,), _f32)
        return carry
    lax.fori_loop(0, RPT2 // 16, _z16, 0)
    for jc in range(3):
        pltpu.sync_copy(ob, spm_cols[jc].at[pl.ds(RPT2 * s, RPT2)])

        @pl.when(s == 0)
        def _():
            pltpu.sync_copy(ob.at[pl.ds(0, 16)],
                            spm_cols[jc].at[pl.ds(NPC, 16)])
    plsc.subcore_barrier()

    gar = jnp.full((16,), NPC, _i32) + lax.iota(_i32, 16)

    def _nk(k, carry):
        jj = k // 8
        oo = (k % 8) * 16
        sv = src_v[jj, pl.ds(oo, 16)]
        dvv = dst_v[jj, pl.ds(oo, 16)]
        ev = ew_v[jj, pl.ds(oo, 16)]
        dsg = plsc.load_gather(dinv_v, [sv])
        ddg = plsc.load_gather(dinv_v, [dvv])
        ew_w = ev * dsg * ddg
        ew_v[jj, pl.ds(oo, 16)] = ew_w
        ld = dvv - NPC * c
        ok = (ld >= 0) & (ld < NPC)
        dst_v[jj, pl.ds(oo, 16)] = jnp.where(ok, ld, gar)
        return carry
    lax.fori_loop(0, CPT * 8, _nk, 0)

    def _ej(j, carry):
        def _sg(g, carry2):
            oo = g * 16
            sv = src_v[j, pl.ds(oo, 16)]
            nm16 = ew_v[j, pl.ds(oo, 16)]
            for jc in range(3):
                zv = plsc.load_gather(z_cols[jc], [sv])
                msg_w[jc, pl.ds(oo, 16)] = zv * nm16
            return carry2
        lax.fori_loop(0, 8, _sg, 0)
        for jc in range(3):
            pltpu.sync_copy(msg_w.at[jc], spm_cols[jc].at[dst_v.at[j]],
                            add=True)
        return carry
    lax.fori_loop(0, CPT, _ej, 0)
    plsc.subcore_barrier()

    b2r = b2_v[pl.ds(0, 16)]
    for jc in range(3):
        pltpu.sync_copy(spm_cols[jc].at[pl.ds(RPT2 * s, RPT2)], ab)
        b2s = _vsplat(b2r, jc)

        def _ei(i, carry):
            oo = i * 16
            zv = z_cols[jc][pl.ds(g0 + oo, 16)]
            dvv = dinv_v[pl.ds(g0 + oo, 16)]
            ob[pl.ds(oo, 16)] = ab[pl.ds(oo, 16)] + dvv * dvv * zv + b2s
            return carry
        lax.fori_loop(0, RPT2 // 16, _ei, 0)
        pltpu.sync_copy(ob, out_hbm.at[pl.ds(jc * NP + g0, RPT2)])


def _make_sc2():
    mesh = plsc.VectorSubcoreMesh(core_axis_name="c", subcore_axis_name="s")
    return pl.kernel(
        _sc2_body,
        out_type=jax.ShapeDtypeStruct((3 * NP,), _f32),
        mesh=mesh,
        compiler_params=pltpu.CompilerParams(needs_layout_passes=False),
        scratch_types=[
            pltpu.VMEM_SHARED((NPC + 16,), _f32),
            pltpu.VMEM_SHARED((NPC + 16,), _f32),
            pltpu.VMEM_SHARED((NPC + 16,), _f32),
            pltpu.VMEM((CPT, CHUNK), _i32),
            pltpu.VMEM((CPT, CHUNK), _i32),
            pltpu.VMEM((CPT, CHUNK), _f32),
            pltpu.VMEM((NP,), _f32),
            pltpu.VMEM((NP,), _f32),
            pltpu.VMEM((NP,), _f32),
            pltpu.VMEM((NP,), _f32),
            pltpu.VMEM((3, CHUNK), _f32),
            pltpu.VMEM((RPT2,), _f32),
            pltpu.VMEM((RPT2,), _f32),
            pltpu.VMEM((16,), _f32),
        ],
    )


def _tc_body(a0_ref, a1_ref, w1a_ref, w1b_ref, b1_ref, w2_ref, out_ref):
    a0 = a0_ref[0]
    a1 = a1_ref[0]
    pre = (jnp.dot(a0, w1a_ref[...], preferred_element_type=_f32)
           + jnp.dot(a1, w1b_ref[...], preferred_element_type=_f32)
           + b1_ref[...])
    h = jnp.maximum(pre, 0.0)
    out_ref[...] = jnp.dot(h, w2_ref[...], preferred_element_type=_f32)


def _tc_call(agg, W1, b1r, W2p):
    blk = 1024
    grid = NP // blk
    return pl.pallas_call(
        _tc_body,
        grid=(grid,),
        in_specs=[
            pl.BlockSpec((1, blk, 128), lambda i: (0, i, 0)),
            pl.BlockSpec((1, blk, 128), lambda i: (1, i, 0)),
            pl.BlockSpec((128, 512), lambda i: (0, 0)),
            pl.BlockSpec((128, 512), lambda i: (1, 0)),
            pl.BlockSpec((1, 512), lambda i: (0, 0)),
            pl.BlockSpec((512, 16), lambda i: (0, 0)),
        ],
        out_specs=pl.BlockSpec((blk, 16), lambda i: (i, 0)),
        out_shape=jax.ShapeDtypeStruct((NP, 16), _f32),
    )(agg, agg, W1, W1, b1r, W2p)


def kernel(x, edge_index, edge_attr, W1, b1, W2, b2):
    src = edge_index[0].astype(_i32)
    dst = edge_index[1].astype(_i32)
    ew = edge_attr.astype(_f32)
    pad = EP - E
    srcr = jnp.pad(src, (0, pad)).reshape(NS, CPT, CHUNK)
    dstr = jnp.pad(dst, (0, pad)).reshape(NS, CPT, CHUNK)
    ewr = jnp.pad(ew, (0, pad)).reshape(NS, CPT, CHUNK)
    xp = jnp.pad(x, ((0, NP - N), (0, 0)))
    xg = jnp.concatenate([xp[:, :128], xp[:, 128:]], axis=0)

    agg, dinv = _make_sc1()(srcr, dstr, ewr, xg)
    z16 = _tc_call(agg, W1, b1.reshape(1, 512), jnp.pad(W2, ((0, 0), (0, 13))))
    zt = z16[:, :3].T.reshape(-1)
    b2r = jnp.pad(b2, (0, 13))
    outt = _make_sc2()(srcr, dstr, ewr, dinv, zt, b2r)
    return outt.reshape(3, NP)[:, :N].T


# speedup vs baseline: 1.0586x; 1.0586x over previous
import functools

import jax
import jax.numpy as jnp
from jax import lax
from jax.experimental import pallas as pl
from jax.experimental.pallas import tpu as pltpu
from jax.experimental.pallas import tpu_sc as plsc

N = 10000
NP = 10240
E = 160000
CHUNK = 128
CPT = 80
ET = CPT * CHUNK
EP = 16 * ET
NC, NS = 2, 16
RPT = NP // NS
NPC = NP // NC
RPT2 = NPC // NS

_i32 = jnp.int32
_f32 = jnp.float32


def _rsqrt16(d):
    h = d * 0.5
    i = lax.bitcast_convert_type(d, _i32)
    i = jnp.full((16,), 0x5F3759DF, _i32) - lax.shift_right_logical(i, 1)
    y = lax.bitcast_convert_type(i, _f32)
    y = y * (1.5 - h * y * y)
    y = y * (1.5 - h * y * y)
    y = y * (1.5 - h * y * y)
    return y


def _vsplat(vec, lane):
    return jnp.take_along_axis(vec, jnp.full((16,), lane, _i32), axis=0,
                               mode="promise_in_bounds")


W1CH = 8


def _sc1_body(src_hbm, dst_hbm, ew_hbm, xg_hbm, xh_hbm, agg_hbm, dinv_hbm,
              spm_agg, spm_deg, spm_dinv,
              src_w, dst_w, ew_w, dinv_v, dv, rows_a, bf_a, bf_b,
              sem_a, sem_b, sem_sa, sem_sb):
    c = lax.axis_index("c")
    s = lax.axis_index("s")
    nw = CPT // W1CH

    def _z16(i, carry):
        dv[pl.ds(i * 16, 16)] = jnp.zeros((16,), _f32)
        return carry
    lax.fori_loop(0, 40, _z16, 0)
    pltpu.sync_copy(dv, spm_deg.at[pl.ds(s * 640, 640)])
    plsc.subcore_barrier()

    def _degw(w, carry):
        pltpu.sync_copy(dst_hbm.at[s, pl.ds(w * W1CH, W1CH)], dst_w)
        pltpu.sync_copy(ew_hbm.at[s, pl.ds(w * W1CH, W1CH)], ew_w)

        def _degj(j, carry2):
            pltpu.sync_copy(ew_w.at[j], spm_deg.at[dst_w.at[j]], add=True)
            return carry2
        lax.fori_loop(0, W1CH, _degj, 0)
        return carry
    lax.fori_loop(0, nw, _degw, 0)
    plsc.subcore_barrier()

    pltpu.sync_copy(spm_deg.at[pl.ds(s * 640, 640)], dv)

    def _dinvi(i, carry):
        d = dv[pl.ds(i * 16, 16)] + 1.0
        dv[pl.ds(i * 16, 16)] = _rsqrt16(d)
        return carry
    lax.fori_loop(0, 40, _dinvi, 0)
    pltpu.sync_copy(dv, spm_dinv.at[pl.ds(s * 640, 640)])

    @pl.when(c == 0)
    def _():
        pltpu.sync_copy(dv, dinv_hbm.at[pl.ds(s * 640, 640)])
    plsc.subcore_barrier()

    pltpu.sync_copy(spm_dinv, dinv_v)

    base = RPT * s
    for t in range(5):
        r0 = base + 128 * t
        pltpu.sync_copy(xg_hbm.at[pl.ds(NP * c + r0, 128)], rows_a)

        def _initr(r, carry):
            g = plsc.load_gather(dinv_v, [jnp.full((16,), r0, _i32) + r])
            g2 = g * g
            for f in range(8):
                rows_a[r, pl.ds(f * 16, 16)] = rows_a[r, pl.ds(f * 16, 16)] * g2
            return carry
        lax.fori_loop(0, 128, _initr, 0)
        pltpu.sync_copy(rows_a, spm_agg.at[pl.ds(r0, 128)])
    plsc.subcore_barrier()

    def _scale(bf, j):
        @plsc.parallel_loop(0, 8, unroll=2)
        def _sg(g):
            nm16 = ew_w[j, pl.ds(g * 16, 16)]
            for l in range(16):
                spl = _vsplat(nm16, l)
                e = g * 16 + l
                for q in range(4):
                    lo, hi = plsc.unpack(bf[e, pl.ds(32 * q, 32)],
                                         format=plsc.PackFormat.INTERLEAVED)
                    rows_a[e, pl.ds(32 * q, 16)] = lo * spl
                    rows_a[e, pl.ds(32 * q + 16, 16)] = hi * spl

    def _gather(j, rows, sem):
        for q in range(4):
            pltpu.async_copy(xh_hbm.at[src_w.at[j, pl.ds(32 * q, 32)]],
                             rows.at[pl.ds(32 * q, 32)], sem)

    def _wait_gather(j, rows, sem):
        for q in range(4):
            pltpu.make_async_copy(xh_hbm.at[src_w.at[j, pl.ds(32 * q, 32)]],
                                  rows.at[pl.ds(32 * q, 32)], sem).wait()

    def _ew(w, carry):
        pltpu.sync_copy(src_hbm.at[s, pl.ds(w * W1CH, W1CH)], src_w)
        pltpu.sync_copy(dst_hbm.at[s, pl.ds(w * W1CH, W1CH)], dst_w)
        pltpu.sync_copy(ew_hbm.at[s, pl.ds(w * W1CH, W1CH)], ew_w)

        def _nk(k, carry2):
            jj = k // 8
            oo = (k % 8) * 16
            sv = src_w[jj, pl.ds(oo, 16)]
            dvv = dst_w[jj, pl.ds(oo, 16)]
            ev = ew_w[jj, pl.ds(oo, 16)]
            dsg = plsc.load_gather(dinv_v, [sv])
            ddg = plsc.load_gather(dinv_v, [dvv])
            ew_w[jj, pl.ds(oo, 16)] = ev * dsg * ddg
            src_w[jj, pl.ds(oo, 16)] = sv + NP * c
            return carry2
        lax.fori_loop(0, W1CH * 8, _nk, 0)

        _gather(0, bf_a, sem_a)
        _gather(1, bf_b, sem_b)

        def _ej(jj, carry2):
            j0 = 2 * jj
            j1 = 2 * jj + 1
            _wait_gather(j0, bf_a, sem_a)
            _scale(bf_a, j0)

            @pl.when(jj < W1CH // 2 - 1)
            def _():
                _gather(j0 + 2, bf_a, sem_a)
            pltpu.sync_copy(rows_a, spm_agg.at[dst_w.at[j0]], add=True)
            _wait_gather(j1, bf_b, sem_b)
            _scale(bf_b, j1)

            @pl.when(jj < W1CH // 2 - 1)
            def _():
                _gather(j1 + 2, bf_b, sem_b)
            pltpu.sync_copy(rows_a, spm_agg.at[dst_w.at[j1]], add=True)
            return carry2
        lax.fori_loop(0, W1CH // 2, _ej, 0)
        return carry
    lax.fori_loop(0, nw, _ew, 0)
    plsc.subcore_barrier()

    pltpu.sync_copy(spm_agg.at[pl.ds(base, RPT)],
                    agg_hbm.at[c, pl.ds(base, RPT)])


def _make_sc1():
    mesh = plsc.VectorSubcoreMesh(core_axis_name="c", subcore_axis_name="s")
    return pl.kernel(
        _sc1_body,
        out_type=(
            jax.ShapeDtypeStruct((NC, NP, 128), _f32),
            jax.ShapeDtypeStruct((NP,), _f32),
        ),
        mesh=mesh,
        compiler_params=pltpu.CompilerParams(needs_layout_passes=False,
                                             use_tc_tiling_on_sc=False),
        scratch_types=[
            pltpu.VMEM_SHARED((NP, 128), _f32),
            pltpu.VMEM_SHARED((NP,), _f32),
            pltpu.VMEM_SHARED((NP,), _f32),
            pltpu.VMEM((W1CH, CHUNK), _i32),
            pltpu.VMEM((W1CH, CHUNK), _i32),
            pltpu.VMEM((W1CH, CHUNK), _f32),
            pltpu.VMEM((NP,), _f32),
            pltpu.VMEM((640,), _f32),
            pltpu.VMEM((CHUNK, 128), _f32),
            pltpu.VMEM((CHUNK, 128), jnp.bfloat16),
            pltpu.VMEM((CHUNK, 128), jnp.bfloat16),
            pltpu.SemaphoreType.DMA,
            pltpu.SemaphoreType.DMA,
            pltpu.SemaphoreType.DMA,
            pltpu.SemaphoreType.DMA,
        ],
    )


def _sc2_body(src_hbm, dst_hbm, ew_hbm, dinv_hbm, zt_hbm, b2_hbm, out_hbm,
              spm_c0, spm_c1, spm_c2,
              src_v, dst_v, ew_v, dinv_v, z0, z1, z2, msg_w, ab, ob, b2_v):
    c = lax.axis_index("c")
    s = lax.axis_index("s")
    g0 = NPC * c + RPT2 * s
    spm_cols = (spm_c0, spm_c1, spm_c2)
    z_cols = (z0, z1, z2)

    pltpu.sync_copy(src_hbm.at[s], src_v)
    pltpu.sync_copy(dst_hbm.at[s], dst_v)
    pltpu.sync_copy(ew_hbm.at[s], ew_v)
    pltpu.sync_copy(dinv_hbm, dinv_v)
    pltpu.sync_copy(b2_hbm, b2_v)
    for jc in range(3):
        pltpu.sync_copy(zt_hbm.at[pl.ds(jc * NP, NP)], z_cols[jc])

    def _z16(i, carry):
        ob[pl.ds(i * 16, 16)] = jnp.zeros((16,), _f32)
        return carry
    lax.fori_loop(0, RPT2 // 16, _z16, 0)
    for jc in range(3):
        pltpu.sync_copy(ob, spm_cols[jc].at[pl.ds(RPT2 * s, RPT2)])

        @pl.when(s == 0)
        def _():
            pltpu.sync_copy(ob.at[pl.ds(0, 16)],
                            spm_cols[jc].at[pl.ds(NPC, 16)])
    plsc.subcore_barrier()

    gar = jnp.full((16,), NPC, _i32) + lax.iota(_i32, 16)

    def _nk(k, carry):
        jj = k // 8
        oo = (k % 8) * 16
        sv = src_v[jj, pl.ds(oo, 16)]
        dvv = dst_v[jj, pl.ds(oo, 16)]
        ev = ew_v[jj, pl.ds(oo, 16)]
        dsg = plsc.load_gather(dinv_v, [sv])
        ddg = plsc.load_gather(dinv_v, [dvv])
        ew_w = ev * dsg * ddg
        ew_v[jj, pl.ds(oo, 16)] = ew_w
        ld = dvv - NPC * c
        ok = (ld >= 0) & (ld < NPC)
        dst_v[jj, pl.ds(oo, 16)] = jnp.where(ok, ld, gar)
        return carry
    lax.fori_loop(0, CPT * 8, _nk, 0)

    def _ej(j, carry):
        def _sg(g, carry2):
            oo = g * 16
            sv = src_v[j, pl.ds(oo, 16)]
            nm16 = ew_v[j, pl.ds(oo, 16)]
            for jc in range(3):
                zv = plsc.load_gather(z_cols[jc], [sv])
                msg_w[jc, pl.ds(oo, 16)] = zv * nm16
            return carry2
        lax.fori_loop(0, 8, _sg, 0)
        for jc in range(3):
            pltpu.sync_copy(msg_w.at[jc], spm_cols[jc].at[dst_v.at[j]],
                            add=True)
        return carry
    lax.fori_loop(0, CPT, _ej, 0)
    plsc.subcore_barrier()

    b2r = b2_v[pl.ds(0, 16)]
    for jc in range(3):
        pltpu.sync_copy(spm_cols[jc].at[pl.ds(RPT2 * s, RPT2)], ab)
        b2s = _vsplat(b2r, jc)

        def _ei(i, carry):
            oo = i * 16
            zv = z_cols[jc][pl.ds(g0 + oo, 16)]
            dvv = dinv_v[pl.ds(g0 + oo, 16)]
            ob[pl.ds(oo, 16)] = ab[pl.ds(oo, 16)] + dvv * dvv * zv + b2s
            return carry
        lax.fori_loop(0, RPT2 // 16, _ei, 0)
        pltpu.sync_copy(ob, out_hbm.at[pl.ds(jc * NP + g0, RPT2)])


def _make_sc2():
    mesh = plsc.VectorSubcoreMesh(core_axis_name="c", subcore_axis_name="s")
    return pl.kernel(
        _sc2_body,
        out_type=jax.ShapeDtypeStruct((3 * NP,), _f32),
        mesh=mesh,
        compiler_params=pltpu.CompilerParams(needs_layout_passes=False),
        scratch_types=[
            pltpu.VMEM_SHARED((NPC + 16,), _f32),
            pltpu.VMEM_SHARED((NPC + 16,), _f32),
            pltpu.VMEM_SHARED((NPC + 16,), _f32),
            pltpu.VMEM((CPT, CHUNK), _i32),
            pltpu.VMEM((CPT, CHUNK), _i32),
            pltpu.VMEM((CPT, CHUNK), _f32),
            pltpu.VMEM((NP,), _f32),
            pltpu.VMEM((NP,), _f32),
            pltpu.VMEM((NP,), _f32),
            pltpu.VMEM((NP,), _f32),
            pltpu.VMEM((3, CHUNK), _f32),
            pltpu.VMEM((RPT2,), _f32),
            pltpu.VMEM((RPT2,), _f32),
            pltpu.VMEM((16,), _f32),
        ],
    )


def _tc_body(a0_ref, a1_ref, w1a_ref, w1b_ref, b1_ref, w2_ref, out_ref):
    a0 = a0_ref[0]
    a1 = a1_ref[0]
    pre = (jnp.dot(a0, w1a_ref[...], preferred_element_type=_f32)
           + jnp.dot(a1, w1b_ref[...], preferred_element_type=_f32)
           + b1_ref[...])
    h = jnp.maximum(pre, 0.0)
    out_ref[...] = jnp.dot(h, w2_ref[...], preferred_element_type=_f32)


def _tc_call(agg, W1, b1r, W2p):
    blk = 1024
    grid = NP // blk
    return pl.pallas_call(
        _tc_body,
        grid=(grid,),
        in_specs=[
            pl.BlockSpec((1, blk, 128), lambda i: (0, i, 0)),
            pl.BlockSpec((1, blk, 128), lambda i: (1, i, 0)),
            pl.BlockSpec((128, 512), lambda i: (0, 0)),
            pl.BlockSpec((128, 512), lambda i: (1, 0)),
            pl.BlockSpec((1, 512), lambda i: (0, 0)),
            pl.BlockSpec((512, 16), lambda i: (0, 0)),
        ],
        out_specs=pl.BlockSpec((blk, 16), lambda i: (i, 0)),
        out_shape=jax.ShapeDtypeStruct((NP, 16), _f32),
    )(agg, agg, W1, W1, b1r, W2p)


def kernel(x, edge_index, edge_attr, W1, b1, W2, b2):
    src = edge_index[0].astype(_i32)
    dst = edge_index[1].astype(_i32)
    ew = edge_attr.astype(_f32)
    pad = EP - E
    srcr = jnp.pad(src, (0, pad)).reshape(NS, CPT, CHUNK)
    dstr = jnp.pad(dst, (0, pad)).reshape(NS, CPT, CHUNK)
    ewr = jnp.pad(ew, (0, pad)).reshape(NS, CPT, CHUNK)
    xp = jnp.pad(x, ((0, NP - N), (0, 0)))
    xg = jnp.concatenate([xp[:, :128], xp[:, 128:]], axis=0)
    xh = (xg.astype(jnp.bfloat16)
          .reshape(2 * NP, 4, 2, 16).transpose(0, 1, 3, 2)
          .reshape(2 * NP, 128))

    agg, dinv = _make_sc1()(srcr, dstr, ewr, xg, xh)
    z16 = _tc_call(agg, W1, b1.reshape(1, 512), jnp.pad(W2, ((0, 0), (0, 13))))
    zt = z16[:, :3].T.reshape(-1)
    b2r = jnp.pad(b2, (0, 13))
    outt = _make_sc2()(srcr, dstr, ewr, dinv, zt, b2r)
    return outt.reshape(3, NP)[:, :N].T


# speedup vs baseline: 1.0722x; 1.0128x over previous
import functools

import jax
import jax.numpy as jnp
from jax import lax
from jax.experimental import pallas as pl
from jax.experimental.pallas import tpu as pltpu
from jax.experimental.pallas import tpu_sc as plsc

N = 10000
NP = 10240
E = 160000
CHUNK = 128
CPT = 80
ET = CPT * CHUNK
EP = 16 * ET
NC, NS = 2, 16
RPT = NP // NS
NPC = NP // NC
RPT2 = NPC // NS

_i32 = jnp.int32
_f32 = jnp.float32


def _rsqrt16(d):
    h = d * 0.5
    i = lax.bitcast_convert_type(d, _i32)
    i = jnp.full((16,), 0x5F3759DF, _i32) - lax.shift_right_logical(i, 1)
    y = lax.bitcast_convert_type(i, _f32)
    y = y * (1.5 - h * y * y)
    y = y * (1.5 - h * y * y)
    y = y * (1.5 - h * y * y)
    return y


def _vsplat(vec, lane):
    return jnp.take_along_axis(vec, jnp.full((16,), lane, _i32), axis=0,
                               mode="promise_in_bounds")


W1CH = 8


def _sc1_body(src_hbm, dst_hbm, ew_hbm, xg_hbm, xh_hbm, agg_hbm, dinv_hbm,
              spm_agg, spm_deg, spm_dinv,
              src_w, dst_w, ew_w, dinv_v, dv, rows_a, bf_a, bf_b,
              sem_a, sem_b, sem_sa, sem_sb):
    c = lax.axis_index("c")
    s = lax.axis_index("s")
    nw = CPT // W1CH

    def _z16(i, carry):
        dv[pl.ds(i * 16, 16)] = jnp.zeros((16,), _f32)
        return carry
    lax.fori_loop(0, 40, _z16, 0)
    pltpu.sync_copy(dv, spm_deg.at[pl.ds(s * 640, 640)])
    plsc.subcore_barrier()

    def _degw(w, carry):
        pltpu.sync_copy(dst_hbm.at[s, pl.ds(w * W1CH, W1CH)], dst_w)
        pltpu.sync_copy(ew_hbm.at[s, pl.ds(w * W1CH, W1CH)], ew_w)

        def _degj(j, carry2):
            pltpu.async_copy(ew_w.at[j], spm_deg.at[dst_w.at[j]], sem_sa,
                             add=True)
            return carry2
        lax.fori_loop(0, W1CH, _degj, 0)

        def _degd(j, carry2):
            pltpu.make_async_copy(ew_w.at[j], spm_deg.at[dst_w.at[j]],
                                  sem_sa).wait()
            return carry2
        lax.fori_loop(0, W1CH, _degd, 0)
        return carry
    lax.fori_loop(0, nw, _degw, 0)
    plsc.subcore_barrier()

    pltpu.sync_copy(spm_deg.at[pl.ds(s * 640, 640)], dv)

    def _dinvi(i, carry):
        d = dv[pl.ds(i * 16, 16)] + 1.0
        dv[pl.ds(i * 16, 16)] = _rsqrt16(d)
        return carry
    lax.fori_loop(0, 40, _dinvi, 0)
    pltpu.sync_copy(dv, spm_dinv.at[pl.ds(s * 640, 640)])

    @pl.when(c == 0)
    def _():
        pltpu.sync_copy(dv, dinv_hbm.at[pl.ds(s * 640, 640)])
    plsc.subcore_barrier()

    pltpu.sync_copy(spm_dinv, dinv_v)

    base = RPT * s
    for t in range(5):
        r0 = base + 128 * t
        pltpu.sync_copy(xg_hbm.at[pl.ds(NP * c + r0, 128)], rows_a)

        def _initr(r, carry):
            g = plsc.load_gather(dinv_v, [jnp.full((16,), r0, _i32) + r])
            g2 = g * g
            for f in range(8):
                rows_a[r, pl.ds(f * 16, 16)] = rows_a[r, pl.ds(f * 16, 16)] * g2
            return carry
        lax.fori_loop(0, 128, _initr, 0)
        pltpu.sync_copy(rows_a, spm_agg.at[pl.ds(r0, 128)])
    plsc.subcore_barrier()

    def _scale(bf, j):
        @plsc.parallel_loop(0, 8, unroll=2)
        def _sg(g):
            nm16 = ew_w[j, pl.ds(g * 16, 16)]
            for l in range(16):
                spl = _vsplat(nm16, l)
                e = g * 16 + l
                for q in range(4):
                    lo, hi = plsc.unpack(bf[e, pl.ds(32 * q, 32)],
                                         format=plsc.PackFormat.INTERLEAVED)
                    rows_a[e, pl.ds(32 * q, 16)] = lo * spl
                    rows_a[e, pl.ds(32 * q + 16, 16)] = hi * spl

    def _gather(j, rows, sem):
        for q in range(4):
            pltpu.async_copy(xh_hbm.at[src_w.at[j, pl.ds(32 * q, 32)]],
                             rows.at[pl.ds(32 * q, 32)], sem)

    def _wait_gather(j, rows, sem):
        for q in range(4):
            pltpu.make_async_copy(xh_hbm.at[src_w.at[j, pl.ds(32 * q, 32)]],
                                  rows.at[pl.ds(32 * q, 32)], sem).wait()

    def _ew(w, carry):
        pltpu.sync_copy(src_hbm.at[s, pl.ds(w * W1CH, W1CH)], src_w)
        pltpu.sync_copy(dst_hbm.at[s, pl.ds(w * W1CH, W1CH)], dst_w)
        pltpu.sync_copy(ew_hbm.at[s, pl.ds(w * W1CH, W1CH)], ew_w)

        def _nk(k, carry2):
            jj = k // 8
            oo = (k % 8) * 16
            sv = src_w[jj, pl.ds(oo, 16)]
            dvv = dst_w[jj, pl.ds(oo, 16)]
            ev = ew_w[jj, pl.ds(oo, 16)]
            dsg = plsc.load_gather(dinv_v, [sv])
            ddg = plsc.load_gather(dinv_v, [dvv])
            ew_w[jj, pl.ds(oo, 16)] = ev * dsg * ddg
            src_w[jj, pl.ds(oo, 16)] = sv + NP * c
            return carry2
        lax.fori_loop(0, W1CH * 8, _nk, 0)

        _gather(0, bf_a, sem_a)
        _gather(1, bf_b, sem_b)

        def _ej(jj, carry2):
            j0 = 2 * jj
            j1 = 2 * jj + 1
            _wait_gather(j0, bf_a, sem_a)
            _scale(bf_a, j0)

            @pl.when(jj < W1CH // 2 - 1)
            def _():
                _gather(j0 + 2, bf_a, sem_a)
            pltpu.sync_copy(rows_a, spm_agg.at[dst_w.at[j0]], add=True)
            _wait_gather(j1, bf_b, sem_b)
            _scale(bf_b, j1)

            @pl.when(jj < W1CH // 2 - 1)
            def _():
                _gather(j1 + 2, bf_b, sem_b)
            pltpu.sync_copy(rows_a, spm_agg.at[dst_w.at[j1]], add=True)
            return carry2
        lax.fori_loop(0, W1CH // 2, _ej, 0)
        return carry
    lax.fori_loop(0, nw, _ew, 0)
    plsc.subcore_barrier()

    pltpu.sync_copy(spm_agg.at[pl.ds(base, RPT)],
                    agg_hbm.at[c, pl.ds(base, RPT)])


def _make_sc1():
    mesh = plsc.VectorSubcoreMesh(core_axis_name="c", subcore_axis_name="s")
    return pl.kernel(
        _sc1_body,
        out_type=(
            jax.ShapeDtypeStruct((NC, NP, 128), _f32),
            jax.ShapeDtypeStruct((NP,), _f32),
        ),
        mesh=mesh,
        compiler_params=pltpu.CompilerParams(needs_layout_passes=False,
                                             use_tc_tiling_on_sc=False),
        scratch_types=[
            pltpu.VMEM_SHARED((NP, 128), _f32),
            pltpu.VMEM_SHARED((NP,), _f32),
            pltpu.VMEM_SHARED((NP,), _f32),
            pltpu.VMEM((W1CH, CHUNK), _i32),
            pltpu.VMEM((W1CH, CHUNK), _i32),
            pltpu.VMEM((W1CH, CHUNK), _f32),
            pltpu.VMEM((NP,), _f32),
            pltpu.VMEM((640,), _f32),
            pltpu.VMEM((CHUNK, 128), _f32),
            pltpu.VMEM((CHUNK, 128), jnp.bfloat16),
            pltpu.VMEM((CHUNK, 128), jnp.bfloat16),
            pltpu.SemaphoreType.DMA,
            pltpu.SemaphoreType.DMA,
            pltpu.SemaphoreType.DMA,
            pltpu.SemaphoreType.DMA,
        ],
    )


def _sc2_body(src_hbm, dst_hbm, ew_hbm, dinv_hbm, zt_hbm, b2_hbm, out_hbm,
              spm_c0, spm_c1, spm_c2,
              src_v, dst_v, ew_v, dinv_v, z0, z1, z2, msg_w, msg_x,
              ab, ob, b2_v, sem_m, sem_x):
    c = lax.axis_index("c")
    s = lax.axis_index("s")
    g0 = NPC * c + RPT2 * s
    spm_cols = (spm_c0, spm_c1, spm_c2)
    z_cols = (z0, z1, z2)

    pltpu.sync_copy(src_hbm.at[s], src_v)
    pltpu.sync_copy(dst_hbm.at[s], dst_v)
    pltpu.sync_copy(ew_hbm.at[s], ew_v)
    pltpu.sync_copy(dinv_hbm, dinv_v)
    pltpu.sync_copy(b2_hbm, b2_v)
    for jc in range(3):
        pltpu.sync_copy(zt_hbm.at[pl.ds(jc * NP, NP)], z_cols[jc])

    def _z16(i, carry):
        ob[pl.ds(i * 16, 16)] = jnp.zeros((16,), _f32)
        return carry
    lax.fori_loop(0, RPT2 // 16, _z16, 0)
    for jc in range(3):
        pltpu.sync_copy(ob, spm_cols[jc].at[pl.ds(RPT2 * s, RPT2)])

        @pl.when(s == 0)
        def _():
            pltpu.sync_copy(ob.at[pl.ds(0, 16)],
                            spm_cols[jc].at[pl.ds(NPC, 16)])
    plsc.subcore_barrier()

    gar = jnp.full((16,), NPC, _i32) + lax.iota(_i32, 16)

    def _nk(k, carry):
        jj = k // 8
        oo = (k % 8) * 16
        sv = src_v[jj, pl.ds(oo, 16)]
        dvv = dst_v[jj, pl.ds(oo, 16)]
        ev = ew_v[jj, pl.ds(oo, 16)]
        dsg = plsc.load_gather(dinv_v, [sv])
        ddg = plsc.load_gather(dinv_v, [dvv])
        ew_w = ev * dsg * ddg
        ew_v[jj, pl.ds(oo, 16)] = ew_w
        ld = dvv - NPC * c
        ok = (ld >= 0) & (ld < NPC)
        dst_v[jj, pl.ds(oo, 16)] = jnp.where(ok, ld, gar)
        return carry
    lax.fori_loop(0, CPT * 8, _nk, 0)

    def _build(mw, j):
        def _sg(g, carry2):
            oo = g * 16
            sv = src_v[j, pl.ds(oo, 16)]
            nm16 = ew_v[j, pl.ds(oo, 16)]
            for jc in range(3):
                zv = plsc.load_gather(z_cols[jc], [sv])
                mw[jc, pl.ds(oo, 16)] = zv * nm16
            return carry2
        lax.fori_loop(0, 8, _sg, 0)

    def _fire3(mw, j, sem):
        for jc in range(3):
            pltpu.async_copy(mw.at[jc], spm_cols[jc].at[dst_v.at[j]], sem,
                             add=True)

    def _drain3(mw, sem):
        for jc in range(3):
            pltpu.make_async_copy(mw.at[jc], spm_cols[jc].at[dst_v.at[0]],
                                  sem).wait()

    def _ej(p, carry):
        j0 = 2 * p
        j1 = 2 * p + 1

        @pl.when(p > 0)
        def _():
            _drain3(msg_w, sem_m)
        _build(msg_w, j0)
        _fire3(msg_w, j0, sem_m)

        @pl.when(p > 0)
        def _():
            _drain3(msg_x, sem_x)
        _build(msg_x, j1)
        _fire3(msg_x, j1, sem_x)
        return carry
    lax.fori_loop(0, CPT // 2, _ej, 0)
    _drain3(msg_w, sem_m)
    _drain3(msg_x, sem_x)
    plsc.subcore_barrier()

    b2r = b2_v[pl.ds(0, 16)]
    for jc in range(3):
        pltpu.sync_copy(spm_cols[jc].at[pl.ds(RPT2 * s, RPT2)], ab)
        b2s = _vsplat(b2r, jc)

        def _ei(i, carry):
            oo = i * 16
            zv = z_cols[jc][pl.ds(g0 + oo, 16)]
            dvv = dinv_v[pl.ds(g0 + oo, 16)]
            ob[pl.ds(oo, 16)] = ab[pl.ds(oo, 16)] + dvv * dvv * zv + b2s
            return carry
        lax.fori_loop(0, RPT2 // 16, _ei, 0)
        pltpu.sync_copy(ob, out_hbm.at[pl.ds(jc * NP + g0, RPT2)])


def _make_sc2():
    mesh = plsc.VectorSubcoreMesh(core_axis_name="c", subcore_axis_name="s")
    return pl.kernel(
        _sc2_body,
        out_type=jax.ShapeDtypeStruct((3 * NP,), _f32),
        mesh=mesh,
        compiler_params=pltpu.CompilerParams(needs_layout_passes=False),
        scratch_types=[
            pltpu.VMEM_SHARED((NPC + 16,), _f32),
            pltpu.VMEM_SHARED((NPC + 16,), _f32),
            pltpu.VMEM_SHARED((NPC + 16,), _f32),
            pltpu.VMEM((CPT, CHUNK), _i32),
            pltpu.VMEM((CPT, CHUNK), _i32),
            pltpu.VMEM((CPT, CHUNK), _f32),
            pltpu.VMEM((NP,), _f32),
            pltpu.VMEM((NP,), _f32),
            pltpu.VMEM((NP,), _f32),
            pltpu.VMEM((NP,), _f32),
            pltpu.VMEM((3, CHUNK), _f32),
            pltpu.VMEM((3, CHUNK), _f32),
            pltpu.VMEM((RPT2,), _f32),
            pltpu.VMEM((RPT2,), _f32),
            pltpu.VMEM((16,), _f32),
            pltpu.SemaphoreType.DMA,
            pltpu.SemaphoreType.DMA,
        ],
    )


def _tc_body(a0_ref, a1_ref, w1a_ref, w1b_ref, b1_ref, w2_ref, out_ref):
    a0 = a0_ref[0]
    a1 = a1_ref[0]
    pre = (jnp.dot(a0, w1a_ref[...], preferred_element_type=_f32)
           + jnp.dot(a1, w1b_ref[...], preferred_element_type=_f32)
           + b1_ref[...])
    h = jnp.maximum(pre, 0.0)
    out_ref[...] = jnp.dot(h, w2_ref[...], preferred_element_type=_f32)


def _tc_call(agg, W1, b1r, W2p):
    blk = 1024
    grid = NP // blk
    return pl.pallas_call(
        _tc_body,
        grid=(grid,),
        in_specs=[
            pl.BlockSpec((1, blk, 128), lambda i: (0, i, 0)),
            pl.BlockSpec((1, blk, 128), lambda i: (1, i, 0)),
            pl.BlockSpec((128, 512), lambda i: (0, 0)),
            pl.BlockSpec((128, 512), lambda i: (1, 0)),
            pl.BlockSpec((1, 512), lambda i: (0, 0)),
            pl.BlockSpec((512, 16), lambda i: (0, 0)),
        ],
        out_specs=pl.BlockSpec((blk, 16), lambda i: (i, 0)),
        out_shape=jax.ShapeDtypeStruct((NP, 16), _f32),
    )(agg, agg, W1, W1, b1r, W2p)


def kernel(x, edge_index, edge_attr, W1, b1, W2, b2):
    src = edge_index[0].astype(_i32)
    dst = edge_index[1].astype(_i32)
    ew = edge_attr.astype(_f32)
    pad = EP - E
    srcr = jnp.pad(src, (0, pad)).reshape(NS, CPT, CHUNK)
    dstr = jnp.pad(dst, (0, pad)).reshape(NS, CPT, CHUNK)
    ewr = jnp.pad(ew, (0, pad)).reshape(NS, CPT, CHUNK)
    xp = jnp.pad(x, ((0, NP - N), (0, 0)))
    xg = jnp.concatenate([xp[:, :128], xp[:, 128:]], axis=0)
    xh = (xg.astype(jnp.bfloat16)
          .reshape(2 * NP, 4, 2, 16).transpose(0, 1, 3, 2)
          .reshape(2 * NP, 128))

    agg, dinv = _make_sc1()(srcr, dstr, ewr, xg, xh)
    z16 = _tc_call(agg, W1, b1.reshape(1, 512), jnp.pad(W2, ((0, 0), (0, 13))))
    zt = z16[:, :3].T.reshape(-1)
    b2r = jnp.pad(b2, (0, 13))
    outt = _make_sc2()(srcr, dstr, ewr, dinv, zt, b2r)
    return outt.reshape(3, NP)[:, :N].T
